# baseline (speedup 1.0000x reference)
import functools

import jax
import jax.numpy as jnp
from jax import lax
from jax.experimental import pallas as pl
from jax.experimental.pallas import tpu as pltpu
from jax.experimental.pallas import tpu_sc as plsc

BATCH = 16384
EMB_D = 64
PAIR_D = 2 * EMB_D
N_PAIR_ROWS = 1000000 // 2
NC = 2
NS = 16
NW = NC * NS
B_PER_W = BATCH // NW


def _gather_body(uidx_hbm, iidx_hbm, ut_hbm, it_hbm, ug_hbm, ig_hbm,
                 idx_u, idx_i, rows_v, sem):
    wid = lax.axis_index("s") * NC + lax.axis_index("c")
    base = wid * B_PER_W
    pltpu.sync_copy(uidx_hbm.at[pl.ds(base, B_PER_W)], idx_u)
    pltpu.sync_copy(iidx_hbm.at[pl.ds(base, B_PER_W)], idx_i)
    pltpu.async_copy(ut_hbm.at[idx_u], rows_v, sem).wait()
    pltpu.sync_copy(rows_v, ug_hbm.at[pl.ds(base, B_PER_W)])
    pltpu.async_copy(it_hbm.at[idx_i], rows_v, sem).wait()
    pltpu.sync_copy(rows_v, ig_hbm.at[pl.ds(base, B_PER_W)])


@functools.cache
def _sc_gather():
    return pl.kernel(
        _gather_body,
        out_type=(
            jax.ShapeDtypeStruct((BATCH, PAIR_D), jnp.float32),
            jax.ShapeDtypeStruct((BATCH, PAIR_D), jnp.float32),
        ),
        mesh=plsc.VectorSubcoreMesh(
            core_axis_name="c", subcore_axis_name="s",
            num_cores=NC, num_subcores=NS),
        scratch_types=[
            pltpu.VMEM((B_PER_W,), jnp.int32),
            pltpu.VMEM((B_PER_W,), jnp.int32),
            pltpu.VMEM((B_PER_W, PAIR_D), jnp.float32),
            pltpu.SemaphoreType.DMA,
        ],
    )


BLK = 2048


def _mlp_body(u_ref, i_ref, a_ref, w1u_ref, w1i_ref, w1a_ref, b1_ref,
              w2_ref, b2_ref, out_ref):
    pu = a_ref[:, 4:5]
    pi = a_ref[:, 5:6]
    u = jnp.where(pu > 0.5, u_ref[:, EMB_D:], u_ref[:, :EMB_D])
    i = jnp.where(pi > 0.5, i_ref[:, EMB_D:], i_ref[:, :EMB_D])
    h = jnp.dot(u, w1u_ref[...], preferred_element_type=jnp.float32)
    h += jnp.dot(i, w1i_ref[...], preferred_element_type=jnp.float32)
    h += jnp.dot(a_ref[...], w1a_ref[...], preferred_element_type=jnp.float32)
    h += b1_ref[...]
    h = jnp.maximum(h, 0.0)
    logits = jnp.sum(h * w2_ref[...], axis=1, keepdims=True) + b2_ref[...]
    out_ref[...] = jax.nn.sigmoid(logits)


def _mlp(u, i, a_pad, w1u, w1i, w1a, b1f, w2row, b2):
    n_blk = BATCH // BLK
    return pl.pallas_call(
        _mlp_body,
        grid=(n_blk,),
        in_specs=[
            pl.BlockSpec((BLK, PAIR_D), lambda j: (j, 0)),
            pl.BlockSpec((BLK, PAIR_D), lambda j: (j, 0)),
            pl.BlockSpec((BLK, 8), lambda j: (j, 0)),
            pl.BlockSpec((EMB_D, 128), lambda j: (0, 0)),
            pl.BlockSpec((EMB_D, 128), lambda j: (0, 0)),
            pl.BlockSpec((8, 128), lambda j: (0, 0)),
            pl.BlockSpec((1, 128), lambda j: (0, 0)),
            pl.BlockSpec((1, 128), lambda j: (0, 0)),
            pl.BlockSpec((1, 1), lambda j: (0, 0)),
        ],
        out_specs=pl.BlockSpec((BLK, 1), lambda j: (j, 0)),
        out_shape=jax.ShapeDtypeStruct((BATCH, 1), jnp.float32),
    )(u, i, a_pad, w1u, w1i, w1a, b1f, w2row, b2)


@jax.jit
def kernel(users, items, audio, user_table, item_table, Wa, ba, W1, b1, W2, b2):
    users = users.astype(jnp.int32)
    items = items.astype(jnp.int32)
    ut2 = user_table.reshape(N_PAIR_ROWS, PAIR_D)
    it2 = item_table.reshape(N_PAIR_ROWS, PAIR_D)
    ug, ig = _sc_gather()(users >> 1, items >> 1, ut2, it2)

    w1u = W1[:EMB_D]
    w1i = W1[EMB_D:2 * EMB_D]
    w1a4 = Wa @ W1[2 * EMB_D:]
    w1a = jnp.zeros((8, 128), jnp.float32).at[:4].set(w1a4)
    b1f = (b1 + ba @ W1[2 * EMB_D:]).reshape(1, 128)
    a_pad = jnp.zeros((BATCH, 8), jnp.float32).at[:, :4].set(audio)
    a_pad = a_pad.at[:, 4].set((users & 1).astype(jnp.float32))
    a_pad = a_pad.at[:, 5].set((items & 1).astype(jnp.float32))
    w2row = W2.reshape(1, 128)
    b2m = b2.reshape(1, 1)

    out = _mlp(ug, ig, a_pad, w1u, w1i, w1a, b1f, w2row, b2m)
    return out[:, 0]


# speedup vs baseline: 1.6638x; 1.6638x over previous
import functools

import jax
import jax.numpy as jnp
from jax import lax
from jax.experimental import pallas as pl
from jax.experimental.pallas import tpu as pltpu
from jax.experimental.pallas import tpu_sc as plsc

BATCH = 16384
EMB_D = 64
NC = 2
NS = 16
NW = NC * NS
B_PER_W = BATCH // NW


def _gather_one(idx_v, tab_hbm, rows_v, sem, out_hbm, base):
    def issue(j, carry):
        vec = idx_v[pl.ds(j, 16)]
        pltpu.async_copy(tab_hbm.at[vec[0]], rows_v.at[j], sem)
        return carry
    lax.fori_loop(0, B_PER_W, issue, 0, unroll=8)
    pltpu.make_async_copy(tab_hbm.at[pl.ds(0, B_PER_W)], rows_v, sem).wait()
    pltpu.sync_copy(rows_v, out_hbm.at[pl.ds(base, B_PER_W)])


def _gather_body(uidx_hbm, iidx_hbm, ut_hbm, it_hbm, ug_hbm, ig_hbm,
                 idx_uv, idx_iv, rows_v, sem):
    wid = lax.axis_index("s") * NC + lax.axis_index("c")
    base = wid * B_PER_W
    pltpu.sync_copy(uidx_hbm.at[pl.ds(base, B_PER_W)],
                    idx_uv.at[pl.ds(0, B_PER_W)])
    pltpu.sync_copy(iidx_hbm.at[pl.ds(base, B_PER_W)],
                    idx_iv.at[pl.ds(0, B_PER_W)])
    _gather_one(idx_uv, ut_hbm, rows_v, sem, ug_hbm, base)
    _gather_one(idx_iv, it_hbm, rows_v, sem, ig_hbm, base)


@functools.cache
def _sc_gather():
    return pl.kernel(
        _gather_body,
        out_type=(
            jax.ShapeDtypeStruct((BATCH, EMB_D), jnp.float32),
            jax.ShapeDtypeStruct((BATCH, EMB_D), jnp.float32),
        ),
        mesh=plsc.VectorSubcoreMesh(
            core_axis_name="c", subcore_axis_name="s",
            num_cores=NC, num_subcores=NS),
        scratch_types=[
            pltpu.VMEM((B_PER_W + 16,), jnp.int32),
            pltpu.VMEM((B_PER_W + 16,), jnp.int32),
            pltpu.VMEM((B_PER_W, EMB_D), jnp.float32),
            pltpu.SemaphoreType.DMA,
        ],
    )


BLK = 2048


def _mlp_body(u_ref, i_ref, a_ref, w1u_ref, w1i_ref, w1a_ref, b1_ref,
              w2_ref, b2_ref, out_ref):
    h = jnp.dot(u_ref[...], w1u_ref[...], preferred_element_type=jnp.float32)
    h += jnp.dot(i_ref[...], w1i_ref[...], preferred_element_type=jnp.float32)
    h += jnp.dot(a_ref[...], w1a_ref[...], preferred_element_type=jnp.float32)
    h += b1_ref[...]
    h = jnp.maximum(h, 0.0)
    logits = jnp.sum(h * w2_ref[...], axis=1, keepdims=True) + b2_ref[...]
    out_ref[...] = jax.nn.sigmoid(logits)


def _mlp(u, i, a_pad, w1u, w1i, w1a, b1f, w2row, b2):
    n_blk = BATCH // BLK
    return pl.pallas_call(
        _mlp_body,
        grid=(n_blk,),
        in_specs=[
            pl.BlockSpec((BLK, EMB_D), lambda j: (j, 0)),
            pl.BlockSpec((BLK, EMB_D), lambda j: (j, 0)),
            pl.BlockSpec((BLK, 8), lambda j: (j, 0)),
            pl.BlockSpec((EMB_D, 128), lambda j: (0, 0)),
            pl.BlockSpec((EMB_D, 128), lambda j: (0, 0)),
            pl.BlockSpec((8, 128), lambda j: (0, 0)),
            pl.BlockSpec((1, 128), lambda j: (0, 0)),
            pl.BlockSpec((1, 128), lambda j: (0, 0)),
            pl.BlockSpec((1, 1), lambda j: (0, 0)),
        ],
        out_specs=pl.BlockSpec((BLK, 1), lambda j: (j, 0)),
        out_shape=jax.ShapeDtypeStruct((BATCH, 1), jnp.float32),
    )(u, i, a_pad, w1u, w1i, w1a, b1f, w2row, b2)


@jax.jit
def kernel(users, items, audio, user_table, item_table, Wa, ba, W1, b1, W2, b2):
    users = users.astype(jnp.int32)
    items = items.astype(jnp.int32)
    ug, ig = _sc_gather()(users, items, user_table, item_table)

    w1u = W1[:EMB_D]
    w1i = W1[EMB_D:2 * EMB_D]
    w1a4 = Wa @ W1[2 * EMB_D:]
    w1a = jnp.zeros((8, 128), jnp.float32).at[:4].set(w1a4)
    b1f = (b1 + ba @ W1[2 * EMB_D:]).reshape(1, 128)
    a_pad = jnp.zeros((BATCH, 8), jnp.float32).at[:, :4].set(audio)
    w2row = W2.reshape(1, 128)
    b2m = b2.reshape(1, 1)

    out = _mlp(ug, ig, a_pad, w1u, w1i, w1a, b1f, w2row, b2m)
    return out[:, 0]
